# Initial kernel scaffold; baseline (speedup 1.0000x reference)
import jax
import jax.numpy as jnp
from jax.experimental import pallas as pl


def kernel(x, edge_index, edge_attr, u, batch, W1, b1, W2, b2):
    raise NotImplementedError("write your pallas kernel here")


# speedup vs baseline: 1.7643x; 1.7643x over previous
import functools

import jax
import jax.numpy as jnp
from jax import lax
from jax.experimental import pallas as pl
from jax.experimental.pallas import tpu as pltpu
from jax.experimental.pallas import tpu_sc as plsc

N = 100000
E = 3200000
F = 16
NC, NS = 2, 16
NW = NC * NS

CHUNK = 1024
KB = CHUNK // 128
TILE_E = 100352
TILE_ROWS = TILE_E // 128
LAST_E = E - (NW - 1) * TILE_E
NFULL = TILE_E // CHUNK
LAST_NFULL = LAST_E // CHUNK

NPAD = 100096
CNT_SL = NPAD // NS
SUM_SL = NPAD // NS
ZC = 3136


def _sc_agg_body(attr_hbm, col2_hbm, sum_hbm, cnt_hbm,
                 sum_acc, cnt_acc, attr_buf, idx_buf, ones_buf, zcnt,
                 sc_sem, cnt_sem):
    c = lax.axis_index("c")
    s = lax.axis_index("s")
    wid = c * NS + s

    def _zrow(i, _):
        attr_buf[i, :] = jnp.zeros((F,), jnp.float32)
        return 0
    lax.fori_loop(0, CHUNK, _zrow, 0)

    def _zcnt(i, _):
        zcnt[pl.ds(i * 16, 16)] = jnp.zeros((16,), jnp.float32)
        return 0
    lax.fori_loop(0, ZC // 16, _zcnt, 0)

    def _ones(i, _):
        ones_buf[pl.ds(i * 16, 16)] = jnp.ones((16,), jnp.float32)
        return 0
    lax.fori_loop(0, 128 // 16, _ones, 0)

    for k in range(6):
        pltpu.sync_copy(attr_buf.at[pl.ds(0, CHUNK)],
                        sum_acc.at[pl.ds(s * SUM_SL + k * CHUNK, CHUNK)])
    pltpu.sync_copy(attr_buf.at[pl.ds(0, SUM_SL - 6 * CHUNK)],
                    sum_acc.at[pl.ds(s * SUM_SL + 6 * CHUNK, SUM_SL - 6 * CHUNK)])
    pltpu.sync_copy(zcnt, cnt_acc.at[pl.ds(s * CNT_SL, ZC)])
    pltpu.sync_copy(zcnt.at[pl.ds(0, CNT_SL - ZC)],
                    cnt_acc.at[pl.ds(s * CNT_SL + ZC, CNT_SL - ZC)])
    plsc.subcore_barrier()

    base = wid * TILE_E
    brow = wid * TILE_ROWS

    def _scatter(nrows):
        cps = []
        for j2 in range(nrows):
            cps.append(pltpu.async_copy(
                attr_buf.at[pl.ds(j2 * 128, 128)],
                sum_acc.at[idx_buf.at[j2]], sc_sem, add=True))
            cps.append(pltpu.async_copy(
                ones_buf, cnt_acc.at[idx_buf.at[j2]], cnt_sem, add=True))
        for cp in cps:
            cp.wait()

    def _chunk(j, _):
        pltpu.sync_copy(attr_hbm.at[pl.ds(base + j * CHUNK, CHUNK)], attr_buf)
        pltpu.sync_copy(col2_hbm.at[pl.ds(brow + j * KB, KB)], idx_buf)
        _scatter(KB)
        return 0

    nfull = jnp.where(wid == NW - 1, LAST_NFULL, NFULL)
    lax.fori_loop(0, nfull, _chunk, 0)

    plsc.subcore_barrier()

    pltpu.sync_copy(sum_acc.at[pl.ds(s * SUM_SL, SUM_SL)],
                    sum_hbm.at[pl.ds(c * NPAD + s * SUM_SL, SUM_SL)])
    pltpu.sync_copy(cnt_acc.at[pl.ds(s * CNT_SL, ZC)], zcnt)
    pltpu.sync_copy(zcnt, cnt_hbm.at[pl.ds(c * NPAD + s * CNT_SL, ZC)])
    pltpu.sync_copy(cnt_acc.at[pl.ds(s * CNT_SL + ZC, CNT_SL - ZC)],
                    zcnt.at[pl.ds(0, CNT_SL - ZC)])
    pltpu.sync_copy(zcnt.at[pl.ds(0, CNT_SL - ZC)],
                    cnt_hbm.at[pl.ds(c * NPAD + s * CNT_SL + ZC, CNT_SL - ZC)])


_sc_agg = pl.kernel(
    _sc_agg_body,
    out_type=(jax.ShapeDtypeStruct((2 * NPAD, F), jnp.float32),
              jax.ShapeDtypeStruct((2 * NPAD,), jnp.float32)),
    mesh=plsc.VectorSubcoreMesh(core_axis_name="c", subcore_axis_name="s"),
    compiler_params=pltpu.CompilerParams(use_tc_tiling_on_sc=False),
    scratch_types=[
        pltpu.VMEM_SHARED((NPAD, F), jnp.float32),
        pltpu.VMEM_SHARED((NPAD,), jnp.float32),
        pltpu.VMEM((CHUNK, F), jnp.float32),
        pltpu.VMEM((KB, 128), jnp.int32),
        pltpu.VMEM((128,), jnp.float32),
        pltpu.VMEM((ZC,), jnp.float32),
        pltpu.SemaphoreType.DMA,
        pltpu.SemaphoreType.DMA,
    ],
)


BLK = 2000


def _mlp_body(sum_ref, cnt_ref, max_ref, u_ref, w1_ref, b1_ref, w2_ref,
              b2_ref, o_ref):
    sm = sum_ref[0] + sum_ref[1]
    cn = cnt_ref[0] + cnt_ref[1]
    mx = max_ref[...]
    out2 = jnp.where(cn > 0, mx, 0.0)
    out3 = sm / jnp.maximum(cn, 1.0)
    ucol = jnp.full((BLK, 1), u_ref[0, 0], jnp.float32)
    zpad = jnp.zeros((BLK, 15), jnp.float32)
    h = jnp.concatenate([sm, out2, out3, ucol, zpad], axis=1)
    a = jnp.dot(h, w1_ref[...], preferred_element_type=jnp.float32)
    a = a + b1_ref[...]
    g = 0.5 * a * (1.0 + lax.erf(a * (2.0 ** -0.5)))
    o = jnp.dot(g, w2_ref[...], preferred_element_type=jnp.float32)
    o_ref[...] = o + b2_ref[...]


def _mlp(sump, cntp, mx, u, w1p, b1, w2, b2):
    grid = (N // BLK,)
    return pl.pallas_call(
        _mlp_body,
        grid=grid,
        in_specs=[
            pl.BlockSpec((2, BLK, F), lambda i: (0, i, 0)),
            pl.BlockSpec((2, BLK, 1), lambda i: (0, i, 0)),
            pl.BlockSpec((BLK, F), lambda i: (i, 0)),
            pl.BlockSpec((1, 1), lambda i: (0, 0)),
            pl.BlockSpec((64, 256), lambda i: (0, 0)),
            pl.BlockSpec((1, 256), lambda i: (0, 0)),
            pl.BlockSpec((256, 128), lambda i: (0, 0)),
            pl.BlockSpec((1, 128), lambda i: (0, 0)),
        ],
        out_specs=pl.BlockSpec((BLK, 128), lambda i: (i, 0)),
        out_shape=jax.ShapeDtypeStruct((N, 128), jnp.float32),
    )(sump, cntp, mx, u, w1p, b1, w2, b2)


def kernel(x, edge_index, edge_attr, u, batch, W1, b1, W2, b2):
    col = edge_index[1]
    col2 = col.reshape(E // 128, 128)
    sump, cntp = _sc_agg(edge_attr, col2)
    sump = sump.reshape(2, NPAD, F)[:, :N]
    cntp = cntp.reshape(2, NPAD)[:, :N, None]
    mx = jax.ops.segment_max(edge_attr, col, num_segments=N)
    w1p = jnp.pad(W1, ((0, 15), (0, 0)))
    out = _mlp(sump, cntp, mx, u, w1p, b1[None, :], W2, b2[None, :])
    return out


# speedup vs baseline: 2.9617x; 1.6786x over previous
import functools

import jax
import jax.numpy as jnp
from jax import lax
from jax.experimental import pallas as pl
from jax.experimental.pallas import tpu as pltpu
from jax.experimental.pallas import tpu_sc as plsc

N = 100000
E = 3200000
F = 16
NC, NS = 2, 16
NW = NC * NS

CHUNK = 1024
KB = CHUNK // 128
TILE_E = 100352
TILE_ROWS = TILE_E // 128
LAST_E = E - (NW - 1) * TILE_E
NFULL = TILE_E // CHUNK
LAST_NFULL = LAST_E // CHUNK

NPAD = 100096
CNT_SL = NPAD // NS
SUM_SL = NPAD // NS
ZC = 3136

ROWS = E // 128
H0_ROWS = 12504
H1_ROWS = ROWS - H0_ROWS
MKB = 8
H0_CHUNKS = H0_ROWS // MKB
H1_CHUNKS = H1_ROWS // MKB
NEG = -3.0e38


def _sc_agg_body(attr_hbm, col2_hbm, sum_hbm, cnt_hbm,
                 sum_acc, cnt_acc, attr_buf, idx_buf, ones_buf, zcnt,
                 sc_sem, cnt_sem):
    c = lax.axis_index("c")
    s = lax.axis_index("s")
    wid = c * NS + s

    def _zrow(i, _):
        attr_buf[i, :] = jnp.zeros((F,), jnp.float32)
        return 0
    lax.fori_loop(0, CHUNK, _zrow, 0)

    def _zcnt(i, _):
        zcnt[pl.ds(i * 16, 16)] = jnp.zeros((16,), jnp.float32)
        return 0
    lax.fori_loop(0, ZC // 16, _zcnt, 0)

    def _ones(i, _):
        ones_buf[pl.ds(i * 16, 16)] = jnp.ones((16,), jnp.float32)
        return 0
    lax.fori_loop(0, 128 // 16, _ones, 0)

    for k in range(6):
        pltpu.sync_copy(attr_buf.at[pl.ds(0, CHUNK)],
                        sum_acc.at[pl.ds(s * SUM_SL + k * CHUNK, CHUNK)])
    pltpu.sync_copy(attr_buf.at[pl.ds(0, SUM_SL - 6 * CHUNK)],
                    sum_acc.at[pl.ds(s * SUM_SL + 6 * CHUNK, SUM_SL - 6 * CHUNK)])
    pltpu.sync_copy(zcnt, cnt_acc.at[pl.ds(s * CNT_SL, ZC)])
    pltpu.sync_copy(zcnt.at[pl.ds(0, CNT_SL - ZC)],
                    cnt_acc.at[pl.ds(s * CNT_SL + ZC, CNT_SL - ZC)])
    plsc.subcore_barrier()

    base = wid * TILE_E
    brow = wid * TILE_ROWS

    def _scatter(nrows):
        cps = []
        for j2 in range(nrows):
            cps.append(pltpu.async_copy(
                attr_buf.at[pl.ds(j2 * 128, 128)],
                sum_acc.at[idx_buf.at[j2]], sc_sem, add=True))
            cps.append(pltpu.async_copy(
                ones_buf, cnt_acc.at[idx_buf.at[j2]], cnt_sem, add=True))
        for cp in cps:
            cp.wait()

    def _chunk(j, _):
        pltpu.sync_copy(attr_hbm.at[pl.ds(base + j * CHUNK, CHUNK)], attr_buf)
        pltpu.sync_copy(col2_hbm.at[pl.ds(brow + j * KB, KB)], idx_buf)
        _scatter(KB)
        return 0

    nfull = jnp.where(wid == NW - 1, LAST_NFULL, NFULL)
    lax.fori_loop(0, nfull, _chunk, 0)

    plsc.subcore_barrier()

    pltpu.sync_copy(sum_acc.at[pl.ds(s * SUM_SL, SUM_SL)],
                    sum_hbm.at[pl.ds(c * NPAD + s * SUM_SL, SUM_SL)])
    pltpu.sync_copy(cnt_acc.at[pl.ds(s * CNT_SL, ZC)], zcnt)
    pltpu.sync_copy(zcnt, cnt_hbm.at[pl.ds(c * NPAD + s * CNT_SL, ZC)])
    pltpu.sync_copy(cnt_acc.at[pl.ds(s * CNT_SL + ZC, CNT_SL - ZC)],
                    zcnt.at[pl.ds(0, CNT_SL - ZC)])
    pltpu.sync_copy(zcnt.at[pl.ds(0, CNT_SL - ZC)],
                    cnt_hbm.at[pl.ds(c * NPAD + s * CNT_SL + ZC, CNT_SL - ZC)])


_sc_agg = pl.kernel(
    _sc_agg_body,
    out_type=(jax.ShapeDtypeStruct((2 * NPAD, F), jnp.float32),
              jax.ShapeDtypeStruct((2 * NPAD,), jnp.float32)),
    mesh=plsc.VectorSubcoreMesh(core_axis_name="c", subcore_axis_name="s"),
    compiler_params=pltpu.CompilerParams(use_tc_tiling_on_sc=False),
    scratch_types=[
        pltpu.VMEM_SHARED((NPAD, F), jnp.float32),
        pltpu.VMEM_SHARED((NPAD,), jnp.float32),
        pltpu.VMEM((CHUNK, F), jnp.float32),
        pltpu.VMEM((KB, 128), jnp.int32),
        pltpu.VMEM((128,), jnp.float32),
        pltpu.VMEM((ZC,), jnp.float32),
        pltpu.SemaphoreType.DMA,
        pltpu.SemaphoreType.DMA,
    ],
)


def _sc_max_body(attrT_hbm, col2_hbm, max_hbm, macc, vbuf, ibuf):
    c = lax.axis_index("c")
    s = lax.axis_index("s")

    def _init(i, _):
        macc[pl.ds(i * 16, 16)] = jnp.full((16,), NEG, jnp.float32)
        return 0
    lax.fori_loop(0, NPAD // 16, _init, 0)

    row0 = c * H0_ROWS
    nch = jnp.where(c == 0, H0_CHUNKS, H1_CHUNKS)

    def _chunk(j, _):
        r0 = row0 + j * MKB
        pltpu.sync_copy(attrT_hbm.at[s, pl.ds(r0, MKB)], vbuf)
        pltpu.sync_copy(col2_hbm.at[pl.ds(r0, MKB)], ibuf)

        def _vreg(v, badmax):
            r = v // 8
            k = v % 8
            i16 = ibuf[r, pl.ds(k * 16, 16)]
            v16 = vbuf[r, pl.ds(k * 16, 16)]
            cur = plsc.load_gather(macc, [i16])
            plsc.store_scatter(macc, [i16], jnp.maximum(cur, v16))
            chk = plsc.load_gather(macc, [i16])
            bad = (chk < v16).astype(jnp.int32)
            return jnp.maximum(badmax, bad)

        badmax = lax.fori_loop(0, 64, _vreg, jnp.zeros((16,), jnp.int32))
        anybad = lax.reduce_max(badmax, axes=(0,))

        @pl.when(anybad > 0)
        def _():
            def _vreg_slow(v, _):
                r = v // 8
                k = v % 8
                i16 = ibuf[r, pl.ds(k * 16, 16)]
                v16 = vbuf[r, pl.ds(k * 16, 16)]

                def _cond(bad):
                    return lax.reduce_max(bad.astype(jnp.int32), axes=(0,)) > 0

                def _body(bad):
                    cur = plsc.load_gather(macc, [i16])
                    plsc.store_scatter(macc, [i16], jnp.maximum(cur, v16),
                                       mask=bad)
                    chk = plsc.load_gather(macc, [i16])
                    return chk < v16

                chk0 = plsc.load_gather(macc, [i16])
                lax.while_loop(_cond, _body, chk0 < v16)
                return 0
            lax.fori_loop(0, 64, _vreg_slow, 0)
        return 0

    lax.fori_loop(0, nch, _chunk, 0)

    obase = (c * NS + s) * NPAD
    pltpu.sync_copy(macc, max_hbm.at[pl.ds(obase, NPAD)])


_sc_max = pl.kernel(
    _sc_max_body,
    out_type=jax.ShapeDtypeStruct((NW * NPAD,), jnp.float32),
    mesh=plsc.VectorSubcoreMesh(core_axis_name="c", subcore_axis_name="s"),
    compiler_params=pltpu.CompilerParams(use_tc_tiling_on_sc=False,
                                         needs_layout_passes=False),
    scratch_types=[
        pltpu.VMEM((NPAD,), jnp.float32),
        pltpu.VMEM((MKB, 128), jnp.float32),
        pltpu.VMEM((MKB, 128), jnp.int32),
    ],
)


TB = 3200


def _tr_body(in_ref, o_ref):
    o_ref[...] = in_ref[...].T


def _transpose(attr):
    return pl.pallas_call(
        _tr_body,
        grid=(E // TB,),
        in_specs=[pl.BlockSpec((TB, F), lambda i: (i, 0))],
        out_specs=pl.BlockSpec((F, TB), lambda i: (0, i)),
        out_shape=jax.ShapeDtypeStruct((F, E), jnp.float32),
    )(attr)


BLK = 2000


def _mlp_body(sum_ref, cnt_ref, max_ref, u_ref, w1_ref, b1_ref, w2_ref,
              b2_ref, o_ref):
    sm = sum_ref[0] + sum_ref[1]
    cn = cnt_ref[0] + cnt_ref[1]
    mx = max_ref[...]
    out2 = jnp.where(cn > 0, mx, 0.0)
    out3 = sm / jnp.maximum(cn, 1.0)
    ucol = jnp.full((BLK, 1), u_ref[0, 0], jnp.float32)
    zpad = jnp.zeros((BLK, 15), jnp.float32)
    h = jnp.concatenate([sm, out2, out3, ucol, zpad], axis=1)
    a = jnp.dot(h, w1_ref[...], preferred_element_type=jnp.float32)
    a = a + b1_ref[...]
    g = 0.5 * a * (1.0 + lax.erf(a * (2.0 ** -0.5)))
    o = jnp.dot(g, w2_ref[...], preferred_element_type=jnp.float32)
    o_ref[...] = o + b2_ref[...]


def _mlp(sump, cntp, mx, u, w1p, b1, w2, b2):
    grid = (N // BLK,)
    return pl.pallas_call(
        _mlp_body,
        grid=grid,
        in_specs=[
            pl.BlockSpec((2, BLK, F), lambda i: (0, i, 0)),
            pl.BlockSpec((2, BLK, 1), lambda i: (0, i, 0)),
            pl.BlockSpec((BLK, F), lambda i: (i, 0)),
            pl.BlockSpec((1, 1), lambda i: (0, 0)),
            pl.BlockSpec((64, 256), lambda i: (0, 0)),
            pl.BlockSpec((1, 256), lambda i: (0, 0)),
            pl.BlockSpec((256, 128), lambda i: (0, 0)),
            pl.BlockSpec((1, 128), lambda i: (0, 0)),
        ],
        out_specs=pl.BlockSpec((BLK, 128), lambda i: (i, 0)),
        out_shape=jax.ShapeDtypeStruct((N, 128), jnp.float32),
    )(sump, cntp, mx, u, w1p, b1, w2, b2)


def kernel(x, edge_index, edge_attr, u, batch, W1, b1, W2, b2):
    col = edge_index[1]
    col2 = col.reshape(E // 128, 128)
    sump, cntp = _sc_agg(edge_attr, col2)
    sump = sump.reshape(2, NPAD, F)[:, :N]
    cntp = cntp.reshape(2, NPAD)[:, :N, None]
    attrT = _transpose(edge_attr).reshape(F, ROWS, 128)
    mx = _sc_max(attrT, col2)
    mx = jnp.max(mx.reshape(2, NS, NPAD), axis=0)[:, :N].T
    w1p = jnp.pad(W1, ((0, 15), (0, 0)))
    out = _mlp(sump, cntp, mx, u, w1p, b1[None, :], W2, b2[None, :])
    return out


# speedup vs baseline: 4.5203x; 1.5263x over previous
import jax
import jax.numpy as jnp
from jax import lax
from jax.experimental import pallas as pl
from jax.experimental.pallas import tpu as pltpu
from jax.experimental.pallas import tpu_sc as plsc

N = 100000
E = 3200000
F = 16
NC, NS = 2, 16
NW = NC * NS

ROWS = E // 128
H0_ROWS = 12504
H1_ROWS = ROWS - H0_ROWS
MKB = 8
H0_CHUNKS = H0_ROWS // MKB
H1_CHUNKS = H1_ROWS // MKB
NPAIR = H1_CHUNKS // 2

TILE_ROWS = 784
LAST_ROWS = ROWS - (NW - 1) * TILE_ROWS
CNT_CHUNKS = TILE_ROWS // MKB
LAST_CNT_CHUNKS = LAST_ROWS // MKB
CNT_PAIRS = CNT_CHUNKS // 2
LAST_CNT_PAIRS = LAST_CNT_CHUNKS // 2

NPAD = 100096
NEG = -3.0e38


def _sc_all_body(attrT_hbm, col2_hbm, sum_hbm, max_hbm, cnt_hbm,
                 acc, vbufA, ibufA, vbufB, ibufB, semA, semB):
    c = lax.axis_index("c")
    s = lax.axis_index("s")
    wid = c * NS + s
    obase = wid * NPAD

    def _fill(value):
        def _f(i, _):
            acc[pl.ds(i * 16, 16)] = jnp.full((16,), value, jnp.float32)
            return 0
        lax.fori_loop(0, NPAD // 16, _f, 0, unroll=8)

    row0 = c * H0_ROWS

    def _startA(r0):
        pltpu.async_copy(attrT_hbm.at[s, pl.ds(r0, MKB)], vbufA, semA)
        pltpu.async_copy(col2_hbm.at[pl.ds(r0, MKB)], ibufA, semA)

    def _startB(r0):
        pltpu.async_copy(attrT_hbm.at[s, pl.ds(r0, MKB)], vbufB, semB)
        pltpu.async_copy(col2_hbm.at[pl.ds(r0, MKB)], ibufB, semB)

    def _waitA():
        pltpu.make_async_copy(attrT_hbm.at[s, pl.ds(row0, MKB)], vbufA,
                              semA).wait()
        pltpu.make_async_copy(col2_hbm.at[pl.ds(row0, MKB)], ibufA,
                              semA).wait()

    def _waitB():
        pltpu.make_async_copy(attrT_hbm.at[s, pl.ds(row0, MKB)], vbufB,
                              semB).wait()
        pltpu.make_async_copy(col2_hbm.at[pl.ds(row0, MKB)], ibufB,
                              semB).wait()

    def _feature_phase(process):
        nch = jnp.where(c == 0, H0_CHUNKS, H1_CHUNKS)
        _startA(row0)

        def _pair(jj, _):
            j1 = 2 * jj + 1
            _startB(row0 + j1 * MKB)
            _waitA()
            process(vbufA, ibufA)

            @pl.when(jj < NPAIR - 1)
            def _():
                _startA(row0 + (j1 + 1) * MKB)
            _waitB()
            process(vbufB, ibufB)
            return 0

        lax.fori_loop(0, NPAIR, _pair, 0)

        @pl.when(nch > 2 * NPAIR)
        def _():
            r0 = row0 + 2 * NPAIR * MKB
            pltpu.sync_copy(attrT_hbm.at[s, pl.ds(r0, MKB)], vbufA)
            pltpu.sync_copy(col2_hbm.at[pl.ds(r0, MKB)], ibufA)
            process(vbufA, ibufA)

    def _proc_sum(vbuf, ibuf):
        def _vreg(v, _):
            r = v // 8
            k = v % 8
            i16 = ibuf[r, pl.ds(k * 16, 16)]
            v16 = vbuf[r, pl.ds(k * 16, 16)]
            plsc.addupdate_scatter(acc, [i16], v16)
            return 0
        lax.fori_loop(0, 64, _vreg, 0, unroll=8)

    _fill(0.0)
    _feature_phase(_proc_sum)
    pltpu.sync_copy(acc, sum_hbm.at[pl.ds(obase, NPAD)])

    def _proc_max(vbuf, ibuf):
        def _vreg(v, badmax):
            r = v // 8
            k = v % 8
            i16 = ibuf[r, pl.ds(k * 16, 16)]
            v16 = vbuf[r, pl.ds(k * 16, 16)]
            cur = plsc.load_gather(acc, [i16])
            plsc.store_scatter(acc, [i16], jnp.maximum(cur, v16))
            chk = plsc.load_gather(acc, [i16])
            return jnp.maximum(badmax, (chk < v16).astype(jnp.int32))

        badmax = lax.fori_loop(0, 64, _vreg, jnp.zeros((16,), jnp.int32),
                               unroll=4)
        anybad = lax.reduce_max(badmax, axes=(0,))

        @pl.when(anybad > 0)
        def _():
            def _vreg_slow(v, _):
                r = v // 8
                k = v % 8
                i16 = ibuf[r, pl.ds(k * 16, 16)]
                v16 = vbuf[r, pl.ds(k * 16, 16)]

                def _cond(bad):
                    return lax.reduce_max(bad.astype(jnp.int32),
                                          axes=(0,)) > 0

                def _body(bad):
                    cur = plsc.load_gather(acc, [i16])
                    plsc.store_scatter(acc, [i16], jnp.maximum(cur, v16),
                                       mask=bad)
                    chk = plsc.load_gather(acc, [i16])
                    return chk < v16

                chk0 = plsc.load_gather(acc, [i16])
                lax.while_loop(_cond, _body, chk0 < v16)
                return 0
            lax.fori_loop(0, 64, _vreg_slow, 0)

    _fill(NEG)
    _feature_phase(_proc_max)
    pltpu.sync_copy(acc, max_hbm.at[pl.ds(obase, NPAD)])

    def _proc_cnt(ibuf):
        ones = jnp.ones((16,), jnp.float32)

        def _vreg(v, _):
            r = v // 8
            k = v % 8
            i16 = ibuf[r, pl.ds(k * 16, 16)]
            plsc.addupdate_scatter(acc, [i16], ones)
            return 0
        lax.fori_loop(0, 64, _vreg, 0, unroll=8)

    _fill(0.0)
    crow = wid * TILE_ROWS
    npc = jnp.where(wid == NW - 1, LAST_CNT_PAIRS, CNT_PAIRS)

    def _cstartA(r0):
        pltpu.async_copy(col2_hbm.at[pl.ds(r0, MKB)], ibufA, semA)

    def _cstartB(r0):
        pltpu.async_copy(col2_hbm.at[pl.ds(r0, MKB)], ibufB, semB)

    def _cwaitA():
        pltpu.make_async_copy(col2_hbm.at[pl.ds(crow, MKB)], ibufA,
                              semA).wait()

    def _cwaitB():
        pltpu.make_async_copy(col2_hbm.at[pl.ds(crow, MKB)], ibufB,
                              semB).wait()

    _cstartA(crow)

    def _cpair(jj, _):
        j1 = 2 * jj + 1
        _cstartB(crow + j1 * MKB)
        _cwaitA()
        _proc_cnt(ibufA)

        @pl.when(jj < npc - 1)
        def _():
            _cstartA(crow + (j1 + 1) * MKB)
        _cwaitB()
        _proc_cnt(ibufB)
        return 0

    lax.fori_loop(0, npc, _cpair, 0)

    @pl.when(wid == NW - 1)
    def _():
        r0 = crow + 2 * LAST_CNT_PAIRS * MKB
        pltpu.sync_copy(col2_hbm.at[pl.ds(r0, MKB)], ibufA)
        _proc_cnt(ibufA)

    pltpu.sync_copy(acc, cnt_hbm.at[pl.ds(obase, NPAD)])


_sc_all = pl.kernel(
    _sc_all_body,
    out_type=(jax.ShapeDtypeStruct((NW * NPAD,), jnp.float32),
              jax.ShapeDtypeStruct((NW * NPAD,), jnp.float32),
              jax.ShapeDtypeStruct((NW * NPAD,), jnp.float32)),
    mesh=plsc.VectorSubcoreMesh(core_axis_name="c", subcore_axis_name="s"),
    compiler_params=pltpu.CompilerParams(use_tc_tiling_on_sc=False,
                                         needs_layout_passes=False),
    scratch_types=[
        pltpu.VMEM((NPAD,), jnp.float32),
        pltpu.VMEM((MKB, 128), jnp.float32),
        pltpu.VMEM((MKB, 128), jnp.int32),
        pltpu.VMEM((MKB, 128), jnp.float32),
        pltpu.VMEM((MKB, 128), jnp.int32),
        pltpu.SemaphoreType.DMA,
        pltpu.SemaphoreType.DMA,
    ],
)


TB = 3200


def _tr_body(in_ref, o_ref):
    o_ref[...] = in_ref[...].T


def _transpose(attr):
    return pl.pallas_call(
        _tr_body,
        grid=(E // TB,),
        in_specs=[pl.BlockSpec((TB, F), lambda i: (i, 0))],
        out_specs=pl.BlockSpec((F, TB), lambda i: (0, i)),
        out_shape=jax.ShapeDtypeStruct((F, E), jnp.float32),
    )(attr)


BLK = 2000


def _mlp_body(sum_ref, cnt_ref, max_ref, u_ref, w1_ref, b1_ref, w2_ref,
              b2_ref, o_ref):
    sm = sum_ref[...]
    cn = cnt_ref[...]
    mx = max_ref[...]
    out2 = jnp.where(cn > 0, mx, 0.0)
    out3 = sm / jnp.maximum(cn, 1.0)
    ucol = jnp.full((BLK, 1), u_ref[0, 0], jnp.float32)
    zpad = jnp.zeros((BLK, 15), jnp.float32)
    h = jnp.concatenate([sm, out2, out3, ucol, zpad], axis=1)
    a = jnp.dot(h, w1_ref[...], preferred_element_type=jnp.float32)
    a = a + b1_ref[...]
    g = 0.5 * a * (1.0 + lax.erf(a * (2.0 ** -0.5)))
    o = jnp.dot(g, w2_ref[...], preferred_element_type=jnp.float32)
    o_ref[...] = o + b2_ref[...]


def _mlp(sm, cnt, mx, u, w1p, b1, w2, b2):
    grid = (N // BLK,)
    return pl.pallas_call(
        _mlp_body,
        grid=grid,
        in_specs=[
            pl.BlockSpec((BLK, F), lambda i: (i, 0)),
            pl.BlockSpec((BLK, 1), lambda i: (i, 0)),
            pl.BlockSpec((BLK, F), lambda i: (i, 0)),
            pl.BlockSpec((1, 1), lambda i: (0, 0)),
            pl.BlockSpec((64, 256), lambda i: (0, 0)),
            pl.BlockSpec((1, 256), lambda i: (0, 0)),
            pl.BlockSpec((256, 128), lambda i: (0, 0)),
            pl.BlockSpec((1, 128), lambda i: (0, 0)),
        ],
        out_specs=pl.BlockSpec((BLK, 128), lambda i: (i, 0)),
        out_shape=jax.ShapeDtypeStruct((N, 128), jnp.float32),
    )(sm, cnt, mx, u, w1p, b1, w2, b2)


def kernel(x, edge_index, edge_attr, u, batch, W1, b1, W2, b2):
    col = edge_index[1]
    col2 = col.reshape(ROWS, 128)
    attrT = _transpose(edge_attr).reshape(F, ROWS, 128)
    sump, maxp, cntp = _sc_all(attrT, col2)
    sm = jnp.sum(sump.reshape(2, NS, NPAD), axis=0)[:, :N].T
    mx = jnp.max(maxp.reshape(2, NS, NPAD), axis=0)[:, :N].T
    cnt = jnp.sum(cntp.reshape(NW, NPAD), axis=0)[:N, None]
    w1p = jnp.pad(W1, ((0, 15), (0, 0)))
    return _mlp(sm, cnt, mx, u, w1p, b1[None, :], W2, b2[None, :])


# speedup vs baseline: 5.0319x; 1.1132x over previous
import jax
import jax.numpy as jnp
from jax import lax
from jax.experimental import pallas as pl
from jax.experimental.pallas import tpu as pltpu
from jax.experimental.pallas import tpu_sc as plsc

N = 100000
E = 3200000
F = 16
NC, NS = 2, 16
NW = NC * NS

ROWS = E // 128
H0_ROWS = 12504
H1_ROWS = ROWS - H0_ROWS
MKB = 8
H0_CHUNKS = H0_ROWS // MKB
H1_CHUNKS = H1_ROWS // MKB
NPAIR = H1_CHUNKS // 2

TILE_ROWS = 784
LAST_ROWS = ROWS - (NW - 1) * TILE_ROWS
CNT_CHUNKS = TILE_ROWS // MKB
LAST_CNT_CHUNKS = LAST_ROWS // MKB
CNT_PAIRS = CNT_CHUNKS // 2
LAST_CNT_PAIRS = LAST_CNT_CHUNKS // 2

NPAD = 100096
NEG = -3.0e38


def _sc_all_body(attrT_hbm, col2_hbm, sum_hbm, max_hbm, cnt_hbm,
                 acc, vbufA, ibufA, vbufB, ibufB, semA, semB):
    c = lax.axis_index("c")
    s = lax.axis_index("s")
    wid = c * NS + s
    obase = wid * NPAD

    def _fill(value):
        @plsc.parallel_loop(0, NPAD // 16, unroll=8)
        def _f(i):
            acc[pl.ds(i * 16, 16)] = jnp.full((16,), value, jnp.float32)

    row0 = c * H0_ROWS

    def _startA(r0):
        pltpu.async_copy(attrT_hbm.at[s, pl.ds(r0, MKB)], vbufA, semA)
        pltpu.async_copy(col2_hbm.at[pl.ds(r0, MKB)], ibufA, semA)

    def _startB(r0):
        pltpu.async_copy(attrT_hbm.at[s, pl.ds(r0, MKB)], vbufB, semB)
        pltpu.async_copy(col2_hbm.at[pl.ds(r0, MKB)], ibufB, semB)

    def _waitA():
        pltpu.make_async_copy(attrT_hbm.at[s, pl.ds(row0, MKB)], vbufA,
                              semA).wait()
        pltpu.make_async_copy(col2_hbm.at[pl.ds(row0, MKB)], ibufA,
                              semA).wait()

    def _waitB():
        pltpu.make_async_copy(attrT_hbm.at[s, pl.ds(row0, MKB)], vbufB,
                              semB).wait()
        pltpu.make_async_copy(col2_hbm.at[pl.ds(row0, MKB)], ibufB,
                              semB).wait()

    def _feature_phase(process):
        nch = jnp.where(c == 0, H0_CHUNKS, H1_CHUNKS)
        _startA(row0)

        def _pair(jj, _):
            j1 = 2 * jj + 1
            _startB(row0 + j1 * MKB)
            _waitA()
            process(vbufA, ibufA)

            @pl.when(jj < NPAIR - 1)
            def _():
                _startA(row0 + (j1 + 1) * MKB)
            _waitB()
            process(vbufB, ibufB)
            return 0

        lax.fori_loop(0, NPAIR, _pair, 0)

        @pl.when(nch > 2 * NPAIR)
        def _():
            r0 = row0 + 2 * NPAIR * MKB
            pltpu.sync_copy(attrT_hbm.at[s, pl.ds(r0, MKB)], vbufA)
            pltpu.sync_copy(col2_hbm.at[pl.ds(r0, MKB)], ibufA)
            process(vbufA, ibufA)

    def _proc_sum(vbuf, ibuf):
        @plsc.parallel_loop(0, 64, unroll=8)
        def _vreg(v):
            r = v // 8
            k = v % 8
            i16 = ibuf[r, pl.ds(k * 16, 16)]
            v16 = vbuf[r, pl.ds(k * 16, 16)]
            plsc.addupdate_scatter(acc, [i16], v16)

    _fill(0.0)
    _feature_phase(_proc_sum)
    pltpu.sync_copy(acc, sum_hbm.at[pl.ds(obase, NPAD)])

    GV = 8

    def _retry_vreg(i16, v16):
        def _cond(bad):
            return lax.reduce_max(bad.astype(jnp.int32), axes=(0,)) > 0

        def _body(bad):
            cur = plsc.load_gather(acc, [i16])
            plsc.store_scatter(acc, [i16], jnp.maximum(cur, v16),
                               mask=bad)
            chk = plsc.load_gather(acc, [i16])
            return chk < v16

        chk0 = plsc.load_gather(acc, [i16])
        lax.while_loop(_cond, _body, chk0 < v16)

    def _proc_max(vbuf, ibuf):
        def _group(g, _):
            r = g // 2
            k0 = (g % 2) * GV
            i16s = [ibuf[r, pl.ds((k0 + t) * 16, 16)] for t in range(GV)]
            v16s = [vbuf[r, pl.ds((k0 + t) * 16, 16)] for t in range(GV)]
            curs = [plsc.load_gather(acc, [i16s[t]]) for t in range(GV)]
            for t in range(GV):
                plsc.store_scatter(acc, [i16s[t]],
                                   jnp.maximum(curs[t], v16s[t]))
            chks = [plsc.load_gather(acc, [i16s[t]]) for t in range(GV)]
            bad = (chks[0] < v16s[0]).astype(jnp.int32)
            for t in range(1, GV):
                bad = jnp.maximum(bad, (chks[t] < v16s[t]).astype(jnp.int32))
            anybad = lax.reduce_max(bad, axes=(0,))

            @pl.when(anybad > 0)
            def _():
                for t in range(GV):
                    _retry_vreg(i16s[t], v16s[t])
            return 0

        lax.fori_loop(0, 64 // GV, _group, 0)

    _fill(NEG)
    _feature_phase(_proc_max)
    pltpu.sync_copy(acc, max_hbm.at[pl.ds(obase, NPAD)])

    def _proc_cnt(ibuf):
        ones = jnp.ones((16,), jnp.float32)

        @plsc.parallel_loop(0, 64, unroll=8)
        def _vreg(v):
            r = v // 8
            k = v % 8
            i16 = ibuf[r, pl.ds(k * 16, 16)]
            plsc.addupdate_scatter(acc, [i16], ones)

    _fill(0.0)
    crow = wid * TILE_ROWS
    npc = jnp.where(wid == NW - 1, LAST_CNT_PAIRS, CNT_PAIRS)

    def _cstartA(r0):
        pltpu.async_copy(col2_hbm.at[pl.ds(r0, MKB)], ibufA, semA)

    def _cstartB(r0):
        pltpu.async_copy(col2_hbm.at[pl.ds(r0, MKB)], ibufB, semB)

    def _cwaitA():
        pltpu.make_async_copy(col2_hbm.at[pl.ds(crow, MKB)], ibufA,
                              semA).wait()

    def _cwaitB():
        pltpu.make_async_copy(col2_hbm.at[pl.ds(crow, MKB)], ibufB,
                              semB).wait()

    _cstartA(crow)

    def _cpair(jj, _):
        j1 = 2 * jj + 1
        _cstartB(crow + j1 * MKB)
        _cwaitA()
        _proc_cnt(ibufA)

        @pl.when(jj < npc - 1)
        def _():
            _cstartA(crow + (j1 + 1) * MKB)
        _cwaitB()
        _proc_cnt(ibufB)
        return 0

    lax.fori_loop(0, npc, _cpair, 0)

    @pl.when(wid == NW - 1)
    def _():
        r0 = crow + 2 * LAST_CNT_PAIRS * MKB
        pltpu.sync_copy(col2_hbm.at[pl.ds(r0, MKB)], ibufA)
        _proc_cnt(ibufA)

    pltpu.sync_copy(acc, cnt_hbm.at[pl.ds(obase, NPAD)])


_sc_all = pl.kernel(
    _sc_all_body,
    out_type=(jax.ShapeDtypeStruct((NW * NPAD,), jnp.float32),
              jax.ShapeDtypeStruct((NW * NPAD,), jnp.float32),
              jax.ShapeDtypeStruct((NW * NPAD,), jnp.float32)),
    mesh=plsc.VectorSubcoreMesh(core_axis_name="c", subcore_axis_name="s"),
    compiler_params=pltpu.CompilerParams(use_tc_tiling_on_sc=False,
                                         needs_layout_passes=False),
    scratch_types=[
        pltpu.VMEM((NPAD,), jnp.float32),
        pltpu.VMEM((MKB, 128), jnp.float32),
        pltpu.VMEM((MKB, 128), jnp.int32),
        pltpu.VMEM((MKB, 128), jnp.float32),
        pltpu.VMEM((MKB, 128), jnp.int32),
        pltpu.SemaphoreType.DMA,
        pltpu.SemaphoreType.DMA,
    ],
)


TB = 3200


def _tr_body(in_ref, o_ref):
    o_ref[...] = in_ref[...].T


def _transpose(attr):
    return pl.pallas_call(
        _tr_body,
        grid=(E // TB,),
        in_specs=[pl.BlockSpec((TB, F), lambda i: (i, 0))],
        out_specs=pl.BlockSpec((F, TB), lambda i: (0, i)),
        out_shape=jax.ShapeDtypeStruct((F, E), jnp.float32),
    )(attr)


BLK = 2000


def _mlp_body(sum_ref, cnt_ref, max_ref, u_ref, w1_ref, b1_ref, w2_ref,
              b2_ref, o_ref):
    sm = sum_ref[...]
    cn = cnt_ref[...]
    mx = max_ref[...]
    out2 = jnp.where(cn > 0, mx, 0.0)
    out3 = sm / jnp.maximum(cn, 1.0)
    ucol = jnp.full((BLK, 1), u_ref[0, 0], jnp.float32)
    zpad = jnp.zeros((BLK, 15), jnp.float32)
    h = jnp.concatenate([sm, out2, out3, ucol, zpad], axis=1)
    a = jnp.dot(h, w1_ref[...], preferred_element_type=jnp.float32)
    a = a + b1_ref[...]
    g = 0.5 * a * (1.0 + lax.erf(a * (2.0 ** -0.5)))
    o = jnp.dot(g, w2_ref[...], preferred_element_type=jnp.float32)
    o_ref[...] = o + b2_ref[...]


def _mlp(sm, cnt, mx, u, w1p, b1, w2, b2):
    grid = (N // BLK,)
    return pl.pallas_call(
        _mlp_body,
        grid=grid,
        in_specs=[
            pl.BlockSpec((BLK, F), lambda i: (i, 0)),
            pl.BlockSpec((BLK, 1), lambda i: (i, 0)),
            pl.BlockSpec((BLK, F), lambda i: (i, 0)),
            pl.BlockSpec((1, 1), lambda i: (0, 0)),
            pl.BlockSpec((64, 256), lambda i: (0, 0)),
            pl.BlockSpec((1, 256), lambda i: (0, 0)),
            pl.BlockSpec((256, 128), lambda i: (0, 0)),
            pl.BlockSpec((1, 128), lambda i: (0, 0)),
        ],
        out_specs=pl.BlockSpec((BLK, 128), lambda i: (i, 0)),
        out_shape=jax.ShapeDtypeStruct((N, 128), jnp.float32),
    )(sm, cnt, mx, u, w1p, b1, w2, b2)


def kernel(x, edge_index, edge_attr, u, batch, W1, b1, W2, b2):
    col = edge_index[1]
    col2 = col.reshape(ROWS, 128)
    attrT = _transpose(edge_attr).reshape(F, ROWS, 128)
    sump, maxp, cntp = _sc_all(attrT, col2)
    sm = jnp.sum(sump.reshape(2, NS, NPAD), axis=0)[:, :N].T
    mx = jnp.max(maxp.reshape(2, NS, NPAD), axis=0)[:, :N].T
    cnt = jnp.sum(cntp.reshape(NW, NPAD), axis=0)[:N, None]
    w1p = jnp.pad(W1, ((0, 15), (0, 0)))
    return _mlp(sm, cnt, mx, u, w1p, b1[None, :], W2, b2[None, :])
